# baseline (speedup 1.0000x reference)
import functools

import jax
import jax.numpy as jnp
from jax import lax
from jax.experimental import pallas as pl
from jax.experimental.pallas import tpu as pltpu
from jax.experimental.pallas import tpu_sc as plsc

_B = 4096
_L = 16
_NC = 2
_NS = 16
_CW = 16


def _sc_embedding_bag(row3, col3, wt2):
    nr = row3.shape[1]
    nch = nr // _CW

    mesh = plsc.VectorSubcoreMesh(core_axis_name="c", subcore_axis_name="s")

    @functools.partial(
        pl.kernel,
        mesh=mesh,
        out_type=jax.ShapeDtypeStruct((_NC, _NS, _B * _L), jnp.float32),
        scratch_types=[
            pltpu.VMEM((2, _CW, 128), jnp.int32),
            pltpu.VMEM((2, _CW, 128), jnp.int32),
            pltpu.VMEM((_CW, 128), jnp.int32),
            pltpu.VMEM((_CW * 128, _L), jnp.float32),
            pltpu.VMEM((_B * _L,), jnp.float32),
            pltpu.SemaphoreType.DMA,
            pltpu.SemaphoreType.DMA,
        ],
        compiler_params=pltpu.CompilerParams(
            needs_layout_passes=False, use_tc_tiling_on_sc=False
        ),
    )
    def body(
        row_hbm, col_hbm, w_hbm, out_hbm, row_v, col_v, cix, gbuf, acc, sem, semi
    ):
        c = lax.axis_index("c")
        s = lax.axis_index("s")
        iota = lax.iota(jnp.int32, _L)
        zeros = jnp.zeros((_L,), jnp.float32)
        gdn = lax.GatherDimensionNumbers(
            offset_dims=(), collapsed_slice_dims=(0,), start_index_map=(0,)
        )

        def lane_splat(vec, j2):
            return lax.gather(
                vec,
                jnp.full((_L, 1), j2, jnp.int32),
                dimension_numbers=gdn,
                slice_sizes=(1,),
                mode=lax.GatherScatterMode.PROMISE_IN_BOUNDS,
            )

        @plsc.parallel_loop(0, _B // 8, unroll=4)
        def zero_step(i):
            for k in range(8):
                acc[pl.ds((i * 8 + k) * _L, _L)] = zeros

        def fire_idx(ch, bsel):
            pltpu.async_copy(
                row_hbm.at[c, pl.ds(ch * _CW, _CW)], row_v.at[bsel], semi
            )
            pltpu.async_copy(
                col_hbm.at[c, pl.ds(ch * _CW, _CW)], col_v.at[bsel], semi
            )

        def wait_idx(bsel):
            pltpu.make_async_copy(
                row_hbm.at[c, pl.ds(0, _CW)], row_v.at[bsel], semi
            ).wait()
            pltpu.make_async_copy(
                col_hbm.at[c, pl.ds(0, _CW)], col_v.at[bsel], semi
            ).wait()

        fire_idx(0, 0)

        def chunk_pair(k, carry):
            for bsel in range(2):
                ch = k * 2 + bsel
                wait_idx(bsel)

                @pl.when(ch + 1 < nch)
                def _prefetch():
                    fire_idx(ch + 1, 1 - bsel)

                @plsc.parallel_loop(0, _CW, unroll=2)
                def col_xform(i):
                    for kk in range(8):
                        sl = pl.ds(kk * 16, 16)
                        cix[i, sl] = col_v[bsel, i, sl] * _NS + s

                for j in range(_CW):
                    pltpu.async_copy(
                        w_hbm.at[cix.at[j]], gbuf.at[pl.ds(j * 128, 128)], sem
                    )

                def sub_step(jj, carry2):
                    for dj in range(4):
                        pltpu.make_async_copy(
                            w_hbm.at[cix.at[0]],
                            gbuf.at[pl.ds((jj * 4 + dj) * 128, 128)],
                            sem,
                        ).wait()

                    @plsc.parallel_loop(0, 32, unroll=16)
                    def pair_groups(g2):
                        gg = jj * 32 + g2
                        r16 = row_v[bsel, gg >> 3, pl.ds((g2 & 7) * 16, 16)] * _L
                        base = gg * 16
                        for j2 in range(16):
                            spl = lane_splat(r16, j2)
                            vals = gbuf[base + j2, :]
                            plsc.addupdate_scatter(acc, [spl + iota], vals)
                    return carry2

                lax.fori_loop(0, _CW // 4, sub_step, 0)
            return carry

        lax.fori_loop(0, nch // 2, chunk_pair, 0)

        pltpu.sync_copy(acc, out_hbm.at[c, s])

    return body(row3, col3, wt2)


def _finish_body(p_ref, b_ref, o_ref):
    q = p_ref[0] + p_ref[1]
    full = jnp.concatenate([q[t] for t in range(_NS)], axis=-1)
    o_ref[...] = jnp.maximum(full + b_ref[...], 0.0)


def kernel(document_mention_indices, W, b):
    idx = document_mention_indices.astype(jnp.int32)
    e = W.shape[0]
    v = W.shape[1]
    nnz = idx.shape[1]
    row3 = idx[0].reshape(_NC, nnz // (_NC * 128), 128)
    col3 = idx[1].reshape(_NC, nnz // (_NC * 128), 128)
    wt2 = W.T.reshape(v * _NS, _L)

    partials = _sc_embedding_bag(row3, col3, wt2)
    partials = partials.reshape(_NC, _NS, _B, _L)

    blk = 512
    return pl.pallas_call(
        _finish_body,
        grid=(_B // blk,),
        in_specs=[
            pl.BlockSpec((_NC, _NS, blk, _L), lambda i: (0, 0, i, 0)),
            pl.BlockSpec((1, e), lambda i: (0, 0)),
        ],
        out_specs=pl.BlockSpec((blk, e), lambda i: (i, 0)),
        out_shape=jax.ShapeDtypeStruct((_B, e), jnp.float32),
    )(partials, b.reshape(1, e))


# speedup vs baseline: 1.0338x; 1.0338x over previous
import functools

import jax
import jax.numpy as jnp
from jax import lax
from jax.experimental import pallas as pl
from jax.experimental.pallas import tpu as pltpu
from jax.experimental.pallas import tpu_sc as plsc

_B = 4096
_L = 16
_NC = 2
_NS = 16
_CW = 16


def _sc_embedding_bag(row3, col3, wt2):
    nr = row3.shape[1]
    nch = nr // _CW

    mesh = plsc.VectorSubcoreMesh(core_axis_name="c", subcore_axis_name="s")

    @functools.partial(
        pl.kernel,
        mesh=mesh,
        out_type=jax.ShapeDtypeStruct((_NC, _NS, _B * _L), jnp.float32),
        scratch_types=[
            pltpu.VMEM((2, _CW, 128), jnp.int32),
            pltpu.VMEM((2, _CW, 128), jnp.int32),
            pltpu.VMEM((_CW, 128), jnp.int32),
            pltpu.VMEM((_CW * 128, _L), jnp.float32),
            pltpu.VMEM((_B * _L,), jnp.float32),
            pltpu.SemaphoreType.DMA,
            pltpu.SemaphoreType.DMA,
        ],
        compiler_params=pltpu.CompilerParams(
            needs_layout_passes=False, use_tc_tiling_on_sc=False
        ),
    )
    def body(
        row_hbm, col_hbm, w_hbm, out_hbm, row_v, col_v, cix, gbuf, acc, sem, semi
    ):
        c = lax.axis_index("c")
        s = lax.axis_index("s")
        iota = lax.iota(jnp.int32, _L)
        zeros = jnp.zeros((_L,), jnp.float32)
        gdn = lax.GatherDimensionNumbers(
            offset_dims=(), collapsed_slice_dims=(0,), start_index_map=(0,)
        )

        def lane_splat(vec, j2):
            return lax.gather(
                vec,
                jnp.full((_L, 1), j2, jnp.int32),
                dimension_numbers=gdn,
                slice_sizes=(1,),
                mode=lax.GatherScatterMode.PROMISE_IN_BOUNDS,
            )

        @plsc.parallel_loop(0, _B // 8, unroll=4)
        def zero_step(i):
            for k in range(8):
                acc[pl.ds((i * 8 + k) * _L, _L)] = zeros

        def fire_idx(ch, bsel):
            pltpu.async_copy(
                row_hbm.at[c, pl.ds(ch * _CW, _CW)], row_v.at[bsel], semi
            )
            pltpu.async_copy(
                col_hbm.at[c, pl.ds(ch * _CW, _CW)], col_v.at[bsel], semi
            )

        def wait_idx(bsel):
            pltpu.make_async_copy(
                row_hbm.at[c, pl.ds(0, _CW)], row_v.at[bsel], semi
            ).wait()
            pltpu.make_async_copy(
                col_hbm.at[c, pl.ds(0, _CW)], col_v.at[bsel], semi
            ).wait()

        fire_idx(0, 0)

        def chunk_pair(k, carry):
            for bsel in range(2):
                ch = k * 2 + bsel
                wait_idx(bsel)

                @pl.when(ch + 1 < nch)
                def _prefetch():
                    fire_idx(ch + 1, 1 - bsel)

                @plsc.parallel_loop(0, _CW, unroll=2)
                def col_xform(i):
                    for kk in range(8):
                        sl = pl.ds(kk * 16, 16)
                        cix[i, sl] = col_v[bsel, i, sl] * _NS + s

                for j in range(_CW):
                    pltpu.async_copy(
                        w_hbm.at[cix.at[j]], gbuf.at[pl.ds(j * 128, 128)], sem
                    )

                def sub_step(jj, carry2):
                    for dj in range(4):
                        pltpu.make_async_copy(
                            w_hbm.at[cix.at[0]],
                            gbuf.at[pl.ds((jj * 4 + dj) * 128, 128)],
                            sem,
                        ).wait()

                    @plsc.parallel_loop(0, 32, unroll=8)
                    def pair_groups(g2):
                        gg = jj * 32 + g2
                        r16 = row_v[bsel, gg >> 3, pl.ds((g2 & 7) * 16, 16)] * _L
                        base = gg * 16
                        for j2 in range(16):
                            spl = lane_splat(r16, j2)
                            vals = gbuf[base + j2, :]
                            plsc.addupdate_scatter(acc, [spl + iota], vals)
                    return carry2

                lax.fori_loop(0, _CW // 4, sub_step, 0)
            return carry

        lax.fori_loop(0, nch // 2, chunk_pair, 0)

        pltpu.sync_copy(acc, out_hbm.at[c, s])

    return body(row3, col3, wt2)


def _finish_body(p_ref, b_ref, o_ref):
    q = p_ref[0] + p_ref[1]
    full = jnp.concatenate([q[t] for t in range(_NS)], axis=-1)
    o_ref[...] = jnp.maximum(full + b_ref[...], 0.0)


def kernel(document_mention_indices, W, b):
    idx = document_mention_indices.astype(jnp.int32)
    e = W.shape[0]
    v = W.shape[1]
    nnz = idx.shape[1]
    row3 = idx[0].reshape(_NC, nnz // (_NC * 128), 128)
    col3 = idx[1].reshape(_NC, nnz // (_NC * 128), 128)
    wt2 = W.T.reshape(v * _NS, _L)

    partials = _sc_embedding_bag(row3, col3, wt2)
    partials = partials.reshape(_NC, _NS, _B, _L)

    blk = 512
    return pl.pallas_call(
        _finish_body,
        grid=(_B // blk,),
        in_specs=[
            pl.BlockSpec((_NC, _NS, blk, _L), lambda i: (0, 0, i, 0)),
            pl.BlockSpec((1, e), lambda i: (0, 0)),
        ],
        out_specs=pl.BlockSpec((blk, e), lambda i: (i, 0)),
        out_shape=jax.ShapeDtypeStruct((_B, e), jnp.float32),
    )(partials, b.reshape(1, e))
